# baseline (speedup 1.0000x reference)
import jax
import jax.numpy as jnp
from jax.experimental import pallas as pl
from jax.experimental.pallas import tpu as pltpu

_NCHUNK = 16


def _dma_copy(src_ref, out_ref, sems):
    S = out_ref.shape[1]
    rows = S // _NCHUNK
    for i in range(_NCHUNK):
        pltpu.make_async_copy(
            src_ref.at[0:1, pl.ds(i * rows, rows), :],
            out_ref.at[:, pl.ds(i * rows, rows), :],
            sems.at[i],
        ).start()
    for i in range(_NCHUNK):
        pltpu.make_async_copy(
            src_ref.at[0:1, pl.ds(i * rows, rows), :],
            out_ref.at[:, pl.ds(i * rows, rows), :],
            sems.at[i],
        ).wait()


def kernel(last_hidden_state, input_ids):
    del input_ids
    B, S, H = last_hidden_state.shape
    out = pl.pallas_call(
        _dma_copy,
        in_specs=[pl.BlockSpec(memory_space=pl.ANY)],
        out_specs=pl.BlockSpec(memory_space=pl.ANY),
        out_shape=jax.ShapeDtypeStruct((1, S, H), last_hidden_state.dtype),
        scratch_shapes=[pltpu.SemaphoreType.DMA((_NCHUNK,))],
    )(last_hidden_state)
    return out


# speedup vs baseline: 34.6672x; 34.6672x over previous
import jax
import jax.numpy as jnp
from jax.experimental import pallas as pl
from jax.experimental.pallas import tpu as pltpu

_ROWS = 512


def _copy_block(src_ref, out_ref):
    out_ref[...] = src_ref[...]


def kernel(last_hidden_state, input_ids):
    del input_ids
    B, S, H = last_hidden_state.shape
    grid = (S // _ROWS,)
    out = pl.pallas_call(
        _copy_block,
        grid=grid,
        in_specs=[pl.BlockSpec((1, _ROWS, H), lambda i: (0, i, 0))],
        out_specs=pl.BlockSpec((1, _ROWS, H), lambda i: (0, i, 0)),
        out_shape=jax.ShapeDtypeStruct((1, S, H), last_hidden_state.dtype),
    )(last_hidden_state)
    return out


# speedup vs baseline: 42.8542x; 1.2362x over previous
import jax
import jax.numpy as jnp
from jax.experimental import pallas as pl
from jax.experimental.pallas import tpu as pltpu

_ROWS = 1024


def _copy_block(src_ref, out_ref):
    out_ref[...] = src_ref[...]


def kernel(last_hidden_state, input_ids):
    del input_ids
    B, S, H = last_hidden_state.shape
    grid = (S // _ROWS,)
    out = pl.pallas_call(
        _copy_block,
        grid=grid,
        in_specs=[pl.BlockSpec((1, _ROWS, H), lambda i: (0, i, 0))],
        out_specs=pl.BlockSpec((1, _ROWS, H), lambda i: (0, i, 0)),
        out_shape=jax.ShapeDtypeStruct((1, S, H), last_hidden_state.dtype),
    )(last_hidden_state)
    return out
